# Initial kernel scaffold; baseline (speedup 1.0000x reference)
import jax
import jax.numpy as jnp
from jax.experimental import pallas as pl


def kernel(word_embedding, spans, tag_embedding, in_proj_w, in_proj_b, out_proj_w, out_proj_b, W1, b1, W2, b2, ln_g, ln_b, Wout, bout):
    raise NotImplementedError("write your pallas kernel here")


# speedup vs baseline: 3.5573x; 3.5573x over previous
import functools

import jax
import jax.numpy as jnp
from jax.experimental import pallas as pl

B, S, H, T, NH, NL = 4, 512, 1024, 8, 16, 9
HF = 512
EPS = 1e-12

_PREC = jax.lax.Precision.HIGHEST


def _dot_t(a, b):
    return jax.lax.dot_general(a, b, (((1,), (1,)), ((), ())),
                               precision=_PREC,
                               preferred_element_type=jnp.float32)


def _tag_kernel(tag_ref, wv_ref, bv_ref, opw_ref, opb_ref, w1_ref, g_ref):
    t = pl.program_id(0)
    te = tag_ref[pl.ds(t, 1), :]
    vh = _dot_t(te, wv_ref[...]) + bv_ref[...]
    attn = _dot_t(vh, opw_ref[...]) + opb_ref[...]
    g_ref[0, :, :] = _dot_t(attn, w1_ref[...])


def _main_kernel(we_ref, st_ref, en_ref, g_ref, b1_ref, w2_ref, b2_ref,
                 lng_ref, lnb_ref, wout_ref, bout_ref, out_ref):
    raw = we_ref[0]
    pos = jax.lax.broadcasted_iota(jnp.int32, (S, T), 0)
    starts = st_ref[0]
    ends = en_ref[0]
    mask = ((pos >= starts) & (pos < ends)).astype(jnp.float32)
    hpre = jnp.dot(mask, g_ref[...], precision=_PREC,
                   preferred_element_type=jnp.float32) + b1_ref[...]
    h = jnp.maximum(hpre, 0.0)
    tagged_out = _dot_t(h, w2_ref[...]) + b2_ref[...]
    cat = jnp.concatenate([raw, tagged_out], axis=-1)
    mu = jnp.mean(cat, axis=-1, keepdims=True)
    var = jnp.mean((cat - mu) ** 2, axis=-1, keepdims=True)
    ln = (cat - mu) * jax.lax.rsqrt(var + EPS) * lng_ref[...] + lnb_ref[...]
    out_ref[0] = _dot_t(ln, wout_ref[...]) + bout_ref[...]


@functools.partial(jax.jit, static_argnums=())
def kernel(word_embedding, spans, tag_embedding, in_proj_w, in_proj_b,
           out_proj_w, out_proj_b, W1, b1, W2, b2, ln_g, ln_b, Wout, bout):
    f32 = jnp.float32
    Wv = in_proj_w[2 * H:]
    bv = in_proj_b[2 * H:].reshape(1, H)
    opb = out_proj_b.reshape(1, H)

    g = pl.pallas_call(
        _tag_kernel,
        grid=(T,),
        in_specs=[
            pl.BlockSpec((T, H), lambda t: (0, 0)),
            pl.BlockSpec((H, H), lambda t: (0, 0)),
            pl.BlockSpec((1, H), lambda t: (0, 0)),
            pl.BlockSpec((H, H), lambda t: (0, 0)),
            pl.BlockSpec((1, H), lambda t: (0, 0)),
            pl.BlockSpec((H, H), lambda t: (0, t)),
        ],
        out_specs=pl.BlockSpec((1, 1, H), lambda t: (t, 0, 0)),
        out_shape=jax.ShapeDtypeStruct((T, 1, H), f32),
    )(tag_embedding.astype(f32), Wv, bv, out_proj_w, opb, W1)
    g = g.reshape(T, H)

    starts = spans[:, :, 0].astype(jnp.int32).reshape(B, 1, T)
    ends = spans[:, :, 1].astype(jnp.int32).reshape(B, 1, T)

    out = pl.pallas_call(
        _main_kernel,
        grid=(B,),
        in_specs=[
            pl.BlockSpec((1, S, H), lambda b: (b, 0, 0)),
            pl.BlockSpec((1, 1, T), lambda b: (b, 0, 0)),
            pl.BlockSpec((1, 1, T), lambda b: (b, 0, 0)),
            pl.BlockSpec((T, H), lambda b: (0, 0)),
            pl.BlockSpec((1, H), lambda b: (0, 0)),
            pl.BlockSpec((HF, H), lambda b: (0, 0)),
            pl.BlockSpec((1, HF), lambda b: (0, 0)),
            pl.BlockSpec((1, H + HF), lambda b: (0, 0)),
            pl.BlockSpec((1, H + HF), lambda b: (0, 0)),
            pl.BlockSpec((NL, H + HF), lambda b: (0, 0)),
            pl.BlockSpec((1, NL), lambda b: (0, 0)),
        ],
        out_specs=pl.BlockSpec((1, S, NL), lambda b: (b, 0, 0)),
        out_shape=jax.ShapeDtypeStruct((B, S, NL), f32),
    )(word_embedding, starts, ends, g, b1.reshape(1, H), W2,
      b2.reshape(1, HF), ln_g.reshape(1, H + HF), ln_b.reshape(1, H + HF),
      Wout, bout.reshape(1, NL))
    return out


# speedup vs baseline: 9.9896x; 2.8082x over previous
import functools

import jax
import jax.numpy as jnp
from jax.experimental import pallas as pl
from jax.experimental.pallas import tpu as pltpu

B, S, H, T, NH, NL = 4, 512, 1024, 8, 16, 9
HF = 512
EPS = 1e-12

def _dot_t(a, b, prec=jax.lax.Precision.DEFAULT):
    return jax.lax.dot_general(a, b, (((1,), (1,)), ((), ())),
                               precision=prec,
                               preferred_element_type=jnp.float32)


def _tag_kernel(tag_ref, wv_ref, bv_ref, opw_ref, opb_ref, w1_ref, g_ref,
                attn_ref):
    t = pl.program_id(0)

    @pl.when(t == 0)
    def _():
        vh = _dot_t(tag_ref[...], wv_ref[...]) + bv_ref[...]
        attn_ref[...] = _dot_t(vh, opw_ref[...]) + opb_ref[...]

    at = attn_ref[pl.ds(t, 1), :]
    g_ref[0, :, :] = _dot_t(at, w1_ref[...])


def _main_kernel(we_ref, st_ref, en_ref, g_ref, b1_ref, w2_ref, b2_ref,
                 lng_ref, lnb_ref, wout_ref, bout_ref, out_ref):
    raw = we_ref[0]
    pos = jax.lax.broadcasted_iota(jnp.int32, (S, T), 0)
    starts = st_ref[0]
    ends = en_ref[0]
    mask = ((pos >= starts) & (pos < ends)).astype(jnp.float32)
    hpre = jnp.dot(mask, g_ref[...],
                   preferred_element_type=jnp.float32) + b1_ref[...]
    h = jnp.maximum(hpre, 0.0)
    tagged_out = _dot_t(h, w2_ref[...]) + b2_ref[...]
    cat = jnp.concatenate([raw, tagged_out], axis=-1)
    mu = jnp.mean(cat, axis=-1, keepdims=True)
    var = jnp.mean((cat - mu) ** 2, axis=-1, keepdims=True)
    ln = (cat - mu) * jax.lax.rsqrt(var + EPS) * lng_ref[...] + lnb_ref[...]
    out_ref[0] = _dot_t(ln, wout_ref[...],
                        prec=jax.lax.Precision.HIGHEST) + bout_ref[...]


@functools.partial(jax.jit, static_argnums=())
def kernel(word_embedding, spans, tag_embedding, in_proj_w, in_proj_b,
           out_proj_w, out_proj_b, W1, b1, W2, b2, ln_g, ln_b, Wout, bout):
    f32 = jnp.float32
    Wv = in_proj_w[2 * H:]
    bv = in_proj_b[2 * H:].reshape(1, H)
    opb = out_proj_b.reshape(1, H)

    g = pl.pallas_call(
        _tag_kernel,
        grid=(T,),
        in_specs=[
            pl.BlockSpec((T, H), lambda t: (0, 0)),
            pl.BlockSpec((H, H), lambda t: (0, 0)),
            pl.BlockSpec((1, H), lambda t: (0, 0)),
            pl.BlockSpec((H, H), lambda t: (0, 0)),
            pl.BlockSpec((1, H), lambda t: (0, 0)),
            pl.BlockSpec((H, H), lambda t: (0, t)),
        ],
        out_specs=pl.BlockSpec((1, 1, H), lambda t: (t, 0, 0)),
        out_shape=jax.ShapeDtypeStruct((T, 1, H), f32),
        scratch_shapes=[pltpu.VMEM((T, H), f32)],
    )(tag_embedding.astype(f32), Wv, bv, out_proj_w, opb, W1)
    g = g.reshape(T, H)

    starts = spans[:, :, 0].astype(jnp.int32).reshape(B, 1, T)
    ends = spans[:, :, 1].astype(jnp.int32).reshape(B, 1, T)

    out = pl.pallas_call(
        _main_kernel,
        grid=(B,),
        in_specs=[
            pl.BlockSpec((1, S, H), lambda b: (b, 0, 0)),
            pl.BlockSpec((1, 1, T), lambda b: (b, 0, 0)),
            pl.BlockSpec((1, 1, T), lambda b: (b, 0, 0)),
            pl.BlockSpec((T, H), lambda b: (0, 0)),
            pl.BlockSpec((1, H), lambda b: (0, 0)),
            pl.BlockSpec((HF, H), lambda b: (0, 0)),
            pl.BlockSpec((1, HF), lambda b: (0, 0)),
            pl.BlockSpec((1, H + HF), lambda b: (0, 0)),
            pl.BlockSpec((1, H + HF), lambda b: (0, 0)),
            pl.BlockSpec((NL, H + HF), lambda b: (0, 0)),
            pl.BlockSpec((1, NL), lambda b: (0, 0)),
        ],
        out_specs=pl.BlockSpec((1, S, NL), lambda b: (b, 0, 0)),
        out_shape=jax.ShapeDtypeStruct((B, S, NL), f32),
    )(word_embedding, starts, ends, g, b1.reshape(1, H), W2,
      b2.reshape(1, HF), ln_g.reshape(1, H + HF), ln_b.reshape(1, H + HF),
      Wout, bout.reshape(1, NL))
    return out


# speedup vs baseline: 12.4253x; 1.2438x over previous
import functools

import jax
import jax.numpy as jnp
from jax.experimental import pallas as pl
from jax.experimental.pallas import tpu as pltpu

B, S, H, T, NH, NL = 4, 512, 1024, 8, 16, 9
HF = 512
EPS = 1e-12

def _dot_t(a, b, prec=jax.lax.Precision.DEFAULT):
    return jax.lax.dot_general(a, b, (((1,), (1,)), ((), ())),
                               precision=prec,
                               preferred_element_type=jnp.float32)


def _tag_kernel(tag_ref, wv_ref, bv_ref, opw_ref, opb_ref, w1_ref, g_ref,
                attn_ref):
    t = pl.program_id(0)

    @pl.when(t == 0)
    def _():
        vh = _dot_t(tag_ref[...], wv_ref[...]) + bv_ref[...]
        attn_ref[...] = _dot_t(vh, opw_ref[...]) + opb_ref[...]

    at = attn_ref[pl.ds(t, 1), :]
    g_ref[0, :, :] = _dot_t(at, w1_ref[...])


def _main_kernel(we_ref, st_ref, en_ref, g_ref, b1_ref, w2_ref, b2_ref,
                 lng_ref, lnb_ref, wout_ref, bout_ref, out_ref):
    raw = we_ref[0]
    pos = jax.lax.broadcasted_iota(jnp.int32, (S, T), 0)
    starts = st_ref[0]
    ends = en_ref[0]
    mask = ((pos >= starts) & (pos < ends)).astype(jnp.bfloat16)
    hpre = jnp.dot(mask, g_ref[...].astype(jnp.bfloat16),
                   preferred_element_type=jnp.float32) + b1_ref[...]
    h = jnp.maximum(hpre, 0.0).astype(jnp.bfloat16)
    tagged_out = (_dot_t(h, w2_ref[...].astype(jnp.bfloat16))
                  + b2_ref[...])
    cat = jnp.concatenate([raw, tagged_out], axis=-1)
    mu = jnp.mean(cat, axis=-1, keepdims=True)
    var = jnp.mean((cat - mu) ** 2, axis=-1, keepdims=True)
    xhat = (cat - mu) * jax.lax.rsqrt(var + EPS)
    sw = wout_ref[...] * lng_ref[...]
    cvec = _dot_t(lnb_ref[...], wout_ref[...]) + bout_ref[...]
    out_ref[0] = _dot_t(xhat, sw) + cvec


@functools.partial(jax.jit, static_argnums=())
def kernel(word_embedding, spans, tag_embedding, in_proj_w, in_proj_b,
           out_proj_w, out_proj_b, W1, b1, W2, b2, ln_g, ln_b, Wout, bout):
    f32 = jnp.float32
    Wv = in_proj_w[2 * H:]
    bv = in_proj_b[2 * H:].reshape(1, H)
    opb = out_proj_b.reshape(1, H)

    g = pl.pallas_call(
        _tag_kernel,
        grid=(T,),
        in_specs=[
            pl.BlockSpec((T, H), lambda t: (0, 0)),
            pl.BlockSpec((H, H), lambda t: (0, 0)),
            pl.BlockSpec((1, H), lambda t: (0, 0)),
            pl.BlockSpec((H, H), lambda t: (0, 0)),
            pl.BlockSpec((1, H), lambda t: (0, 0)),
            pl.BlockSpec((H, H), lambda t: (0, t)),
        ],
        out_specs=pl.BlockSpec((1, 1, H), lambda t: (t, 0, 0)),
        out_shape=jax.ShapeDtypeStruct((T, 1, H), f32),
        scratch_shapes=[pltpu.VMEM((T, H), f32)],
    )(tag_embedding.astype(f32), Wv, bv, out_proj_w, opb, W1)
    g = g.reshape(T, H)

    starts = spans[:, :, 0].astype(jnp.int32).reshape(B, 1, T)
    ends = spans[:, :, 1].astype(jnp.int32).reshape(B, 1, T)

    out = pl.pallas_call(
        _main_kernel,
        grid=(B,),
        in_specs=[
            pl.BlockSpec((1, S, H), lambda b: (b, 0, 0)),
            pl.BlockSpec((1, 1, T), lambda b: (b, 0, 0)),
            pl.BlockSpec((1, 1, T), lambda b: (b, 0, 0)),
            pl.BlockSpec((T, H), lambda b: (0, 0)),
            pl.BlockSpec((1, H), lambda b: (0, 0)),
            pl.BlockSpec((HF, H), lambda b: (0, 0)),
            pl.BlockSpec((1, HF), lambda b: (0, 0)),
            pl.BlockSpec((1, H + HF), lambda b: (0, 0)),
            pl.BlockSpec((1, H + HF), lambda b: (0, 0)),
            pl.BlockSpec((NL, H + HF), lambda b: (0, 0)),
            pl.BlockSpec((1, NL), lambda b: (0, 0)),
        ],
        out_specs=pl.BlockSpec((1, S, NL), lambda b: (b, 0, 0)),
        out_shape=jax.ShapeDtypeStruct((B, S, NL), f32),
    )(word_embedding, starts, ends, g, b1.reshape(1, H), W2,
      b2.reshape(1, HF), ln_g.reshape(1, H + HF), ln_b.reshape(1, H + HF),
      Wout, bout.reshape(1, NL))
    return out


# speedup vs baseline: 12.5559x; 1.0105x over previous
import functools

import jax
import jax.numpy as jnp
from jax.experimental import pallas as pl
from jax.experimental.pallas import tpu as pltpu

B, S, H, T, NH, NL = 4, 512, 1024, 8, 16, 9
HF = 512
EPS = 1e-12


def _dot_t(a, b):
    return jax.lax.dot_general(a, b, (((1,), (1,)), ((), ())),
                               preferred_element_type=jnp.float32)


def _fused_kernel(tag_ref, wv_ref, bv_ref, opw_ref, opb_ref, w1_ref,
                  we_ref, st_ref, en_ref, b1_ref, w2_ref, b2_ref,
                  lng_ref, lnb_ref, wout_ref, bout_ref,
                  out_ref, attn_ref, g_ref):
    i = pl.program_id(0)

    @pl.when(i == 0)
    def _():
        vh = _dot_t(tag_ref[...], wv_ref[...]) + bv_ref[...]
        attn_ref[...] = _dot_t(vh, opw_ref[...]) + opb_ref[...]

    @pl.when(i < T)
    def _():
        at = attn_ref[pl.ds(i, 1), :]
        g_ref[pl.ds(i, 1), :] = _dot_t(at, w1_ref[...])

    @pl.when(i >= T)
    def _():
        raw = we_ref[0]
        pos = jax.lax.broadcasted_iota(jnp.int32, (S, T), 0)
        starts = st_ref[0]
        ends = en_ref[0]
        mask = ((pos >= starts) & (pos < ends)).astype(jnp.bfloat16)
        hpre = jnp.dot(mask, g_ref[...].astype(jnp.bfloat16),
                       preferred_element_type=jnp.float32) + b1_ref[...]
        h = jnp.maximum(hpre, 0.0).astype(jnp.bfloat16)
        tagged_out = (_dot_t(h, w2_ref[...].astype(jnp.bfloat16))
                      + b2_ref[...])
        cat = jnp.concatenate([raw, tagged_out], axis=-1)
        mu = jnp.mean(cat, axis=-1, keepdims=True)
        var = jnp.mean((cat - mu) ** 2, axis=-1, keepdims=True)
        xhat = (cat - mu) * jax.lax.rsqrt(var + EPS)
        sw = wout_ref[...] * lng_ref[...]
        cvec = _dot_t(lnb_ref[...], wout_ref[...]) + bout_ref[...]
        out_ref[0] = _dot_t(xhat, sw) + cvec


@functools.partial(jax.jit, static_argnums=())
def kernel(word_embedding, spans, tag_embedding, in_proj_w, in_proj_b,
           out_proj_w, out_proj_b, W1, b1, W2, b2, ln_g, ln_b, Wout, bout):
    f32 = jnp.float32
    Wv = in_proj_w[2 * H:]
    bv = in_proj_b[2 * H:].reshape(1, H)
    opb = out_proj_b.reshape(1, H)
    starts = spans[:, :, 0].astype(jnp.int32).reshape(B, 1, T)
    ends = spans[:, :, 1].astype(jnp.int32).reshape(B, 1, T)

    const = lambda i: (0, 0)
    batch3 = lambda i: (jnp.maximum(i - T, 0), 0, 0)

    out = pl.pallas_call(
        _fused_kernel,
        grid=(T + B,),
        in_specs=[
            pl.BlockSpec((T, H), const),
            pl.BlockSpec((H, H), const),
            pl.BlockSpec((1, H), const),
            pl.BlockSpec((H, H), const),
            pl.BlockSpec((1, H), const),
            pl.BlockSpec((H, H), lambda i: (0, jnp.minimum(i, T - 1))),
            pl.BlockSpec((1, S, H), batch3),
            pl.BlockSpec((1, 1, T), batch3),
            pl.BlockSpec((1, 1, T), batch3),
            pl.BlockSpec((1, H), const),
            pl.BlockSpec((HF, H), const),
            pl.BlockSpec((1, HF), const),
            pl.BlockSpec((1, H + HF), const),
            pl.BlockSpec((1, H + HF), const),
            pl.BlockSpec((NL, H + HF), const),
            pl.BlockSpec((1, NL), const),
        ],
        out_specs=pl.BlockSpec((1, S, NL), batch3),
        out_shape=jax.ShapeDtypeStruct((B, S, NL), f32),
        scratch_shapes=[pltpu.VMEM((T, H), f32), pltpu.VMEM((T, H), f32)],
    )(tag_embedding.astype(f32), Wv, bv, out_proj_w, opb, W1,
      word_embedding, starts, ends, b1.reshape(1, H), W2,
      b2.reshape(1, HF), ln_g.reshape(1, H + HF), ln_b.reshape(1, H + HF),
      Wout, bout.reshape(1, NL))
    return out


# speedup vs baseline: 13.4499x; 1.0712x over previous
import functools

import jax
import jax.numpy as jnp
from jax.experimental import pallas as pl
from jax.experimental.pallas import tpu as pltpu

B, S, H, T, NH, NL = 4, 512, 1024, 8, 16, 9
HF = 512
EPS = 1e-12
SB = 256
NSB = S // SB


def _dot_t(a, b):
    return jax.lax.dot_general(a, b, (((1,), (1,)), ((), ())),
                               preferred_element_type=jnp.float32)


def _fused_kernel(tag_ref, wv_ref, bv_ref, opw_ref, opb_ref, w1_ref,
                  we_ref, st_ref, en_ref, b1_ref, w2_ref, b2_ref,
                  lng_ref, lnb_ref, wout_ref, bout_ref,
                  out_ref, attn_ref, g_ref):
    i = pl.program_id(0)

    @pl.when(i == 0)
    def _():
        vh = _dot_t(tag_ref[...], wv_ref[...]) + bv_ref[...]
        attn_ref[...] = _dot_t(vh, opw_ref[...]) + opb_ref[...]

    @pl.when(i < T)
    def _():
        at = attn_ref[pl.ds(i, 1), :]
        g_ref[pl.ds(i, 1), :] = _dot_t(at, w1_ref[...])

    @pl.when(i >= T)
    def _():
        raw = we_ref[0]
        base = (i - T) % (S // SB) * SB
        pos = jax.lax.broadcasted_iota(jnp.int32, (SB, T), 0) + base
        starts = st_ref[0]
        ends = en_ref[0]
        mask = ((pos >= starts) & (pos < ends)).astype(jnp.bfloat16)
        hpre = jnp.dot(mask, g_ref[...].astype(jnp.bfloat16),
                       preferred_element_type=jnp.float32) + b1_ref[...]
        h = jnp.maximum(hpre, 0.0).astype(jnp.bfloat16)
        tagged_out = (_dot_t(h, w2_ref[...].astype(jnp.bfloat16))
                      + b2_ref[...])
        cat = jnp.concatenate([raw, tagged_out], axis=-1)
        mu = jnp.mean(cat, axis=-1, keepdims=True)
        var = jnp.mean((cat - mu) ** 2, axis=-1, keepdims=True)
        xhat = (cat - mu) * jax.lax.rsqrt(var + EPS)
        sw = wout_ref[...] * lng_ref[...]
        cvec = _dot_t(lnb_ref[...], wout_ref[...]) + bout_ref[...]
        out_ref[0] = _dot_t(xhat, sw) + cvec


@functools.partial(jax.jit, static_argnums=())
def kernel(word_embedding, spans, tag_embedding, in_proj_w, in_proj_b,
           out_proj_w, out_proj_b, W1, b1, W2, b2, ln_g, ln_b, Wout, bout):
    f32 = jnp.float32
    bv = in_proj_b[2 * H:].reshape(1, H)
    opb = out_proj_b.reshape(1, H)
    starts = spans[:, :, 0].astype(jnp.int32).reshape(B, 1, T)
    ends = spans[:, :, 1].astype(jnp.int32).reshape(B, 1, T)

    const = lambda i: (0, 0)
    wv_map = lambda i: (2, 0)
    bmap = lambda i: (jnp.maximum(i - T, 0) // NSB, 0, 0)
    rmap = lambda i: (jnp.maximum(i - T, 0) // NSB,
                      jnp.maximum(i - T, 0) % NSB, 0)

    out = pl.pallas_call(
        _fused_kernel,
        grid=(T + B * NSB,),
        in_specs=[
            pl.BlockSpec((T, H), const),
            pl.BlockSpec((H, H), wv_map),
            pl.BlockSpec((1, H), const),
            pl.BlockSpec((H, H), const),
            pl.BlockSpec((1, H), const),
            pl.BlockSpec((H, H), lambda i: (0, jnp.minimum(i, T - 1))),
            pl.BlockSpec((1, SB, H), rmap),
            pl.BlockSpec((1, 1, T), bmap),
            pl.BlockSpec((1, 1, T), bmap),
            pl.BlockSpec((1, H), const),
            pl.BlockSpec((HF, H), const),
            pl.BlockSpec((1, HF), const),
            pl.BlockSpec((1, H + HF), const),
            pl.BlockSpec((1, H + HF), const),
            pl.BlockSpec((NL, H + HF), const),
            pl.BlockSpec((1, NL), const),
        ],
        out_specs=pl.BlockSpec((1, SB, NL), rmap),
        out_shape=jax.ShapeDtypeStruct((B, S, NL), f32),
        scratch_shapes=[pltpu.VMEM((T, H), f32), pltpu.VMEM((T, H), f32)],
    )(tag_embedding.astype(f32), in_proj_w, bv, out_proj_w, opb, W1,
      word_embedding, starts, ends, b1.reshape(1, H), W2,
      b2.reshape(1, HF), ln_g.reshape(1, H + HF), ln_b.reshape(1, H + HF),
      Wout, bout.reshape(1, NL))
    return out


# speedup vs baseline: 13.4583x; 1.0006x over previous
import functools

import jax
import jax.numpy as jnp
from jax.experimental import pallas as pl
from jax.experimental.pallas import tpu as pltpu

B, S, H, T, NH, NL = 4, 512, 1024, 8, 16, 9
HF = 512
EPS = 1e-12
SB = 256
NSB = S // SB
TPB = 2
NTP = T // TPB


def _dot_t(a, b):
    return jax.lax.dot_general(a, b, (((1,), (1,)), ((), ())),
                               preferred_element_type=jnp.float32)


def _fused_kernel(tag_ref, wv_ref, bv_ref, opw_ref, opb_ref, w1_ref,
                  we_ref, st_ref, en_ref, b1_ref, w2_ref, b2_ref,
                  lng_ref, lnb_ref, wout_ref, bout_ref,
                  out_ref, attn_ref, g_ref):
    i = pl.program_id(0)

    @pl.when(i == 0)
    def _():
        vh = _dot_t(tag_ref[...], wv_ref[...]) + bv_ref[...]
        attn_ref[...] = _dot_t(vh, opw_ref[...]) + opb_ref[...]

    @pl.when(i < NTP)
    def _():
        for k in range(TPB):
            t = i * TPB + k
            at = attn_ref[pl.ds(t, 1), :]
            g_ref[pl.ds(t, 1), :] = _dot_t(at, w1_ref[:, k * H:(k + 1) * H])

    @pl.when(i >= NTP)
    def _():
        raw = we_ref[0]
        base = (i - NTP) % NSB * SB
        pos = jax.lax.broadcasted_iota(jnp.int32, (SB, T), 0) + base
        starts = st_ref[0]
        ends = en_ref[0]
        mask = ((pos >= starts) & (pos < ends)).astype(jnp.bfloat16)
        hpre = jnp.dot(mask, g_ref[...].astype(jnp.bfloat16),
                       preferred_element_type=jnp.float32) + b1_ref[...]
        h = jnp.maximum(hpre, 0.0).astype(jnp.bfloat16)
        tagged_out = (_dot_t(h, w2_ref[...].astype(jnp.bfloat16))
                      + b2_ref[...])
        cat = jnp.concatenate([raw, tagged_out], axis=-1)
        mu = jnp.mean(cat, axis=-1, keepdims=True)
        var = jnp.mean((cat - mu) ** 2, axis=-1, keepdims=True)
        xhat = (cat - mu) * jax.lax.rsqrt(var + EPS)
        sw = wout_ref[...] * lng_ref[...]
        cvec = _dot_t(lnb_ref[...], wout_ref[...]) + bout_ref[...]
        out_ref[0] = _dot_t(xhat, sw) + cvec


@functools.partial(jax.jit, static_argnums=())
def kernel(word_embedding, spans, tag_embedding, in_proj_w, in_proj_b,
           out_proj_w, out_proj_b, W1, b1, W2, b2, ln_g, ln_b, Wout, bout):
    f32 = jnp.float32
    bv = in_proj_b[2 * H:].reshape(1, H)
    opb = out_proj_b.reshape(1, H)
    starts = spans[:, :, 0].astype(jnp.int32).reshape(B, 1, T)
    ends = spans[:, :, 1].astype(jnp.int32).reshape(B, 1, T)

    const = lambda i: (0, 0)
    wv_map = lambda i: (2, 0)
    bmap = lambda i: (jnp.maximum(i - NTP, 0) // NSB, 0, 0)
    rmap = lambda i: (jnp.maximum(i - NTP, 0) // NSB,
                      jnp.maximum(i - NTP, 0) % NSB, 0)

    out = pl.pallas_call(
        _fused_kernel,
        grid=(NTP + B * NSB,),
        in_specs=[
            pl.BlockSpec((T, H), const),
            pl.BlockSpec((H, H), wv_map),
            pl.BlockSpec((1, H), const),
            pl.BlockSpec((H, H), const),
            pl.BlockSpec((1, H), const),
            pl.BlockSpec((H, TPB * H), lambda i: (0, jnp.minimum(i, NTP - 1))),
            pl.BlockSpec((1, SB, H), rmap),
            pl.BlockSpec((1, 1, T), bmap),
            pl.BlockSpec((1, 1, T), bmap),
            pl.BlockSpec((1, H), const),
            pl.BlockSpec((HF, H), const),
            pl.BlockSpec((1, HF), const),
            pl.BlockSpec((1, H + HF), const),
            pl.BlockSpec((1, H + HF), const),
            pl.BlockSpec((NL, H + HF), const),
            pl.BlockSpec((1, NL), const),
        ],
        out_specs=pl.BlockSpec((1, SB, NL), rmap),
        out_shape=jax.ShapeDtypeStruct((B, S, NL), f32),
        scratch_shapes=[pltpu.VMEM((T, H), f32), pltpu.VMEM((T, H), f32)],
    )(tag_embedding.astype(f32), in_proj_w, bv, out_proj_w, opb, W1,
      word_embedding, starts, ends, b1.reshape(1, H), W2,
      b2.reshape(1, HF), ln_g.reshape(1, H + HF), ln_b.reshape(1, H + HF),
      Wout, bout.reshape(1, NL))
    return out


# speedup vs baseline: 14.8749x; 1.1053x over previous
import functools

import jax
import jax.numpy as jnp
from jax.experimental import pallas as pl
from jax.experimental.pallas import tpu as pltpu

B, S, H, T, NH, NL = 4, 512, 1024, 8, 16, 9
HF = 512
EPS = 1e-12
SB = 256
NSB = S // SB
TPB = 2
NTP = T // TPB


def _dot_t(a, b):
    return jax.lax.dot_general(a, b, (((1,), (1,)), ((), ())),
                               preferred_element_type=jnp.float32)


def _fused_kernel(tag_ref, wv_ref, bv_ref, opw_ref, opb_ref, w1_ref,
                  we_ref, st_ref, en_ref, b1_ref, w2_ref, b2_ref,
                  lng_ref, lnb_ref, wout_ref, bout_ref,
                  out_ref, attn_ref, g_ref):
    i = pl.program_id(0)

    @pl.when(i == 0)
    def _():
        vh = _dot_t(tag_ref[...], wv_ref[...]) + bv_ref[...]
        attn_ref[...] = _dot_t(vh, opw_ref[...]) + opb_ref[...]

    @pl.when(i < NTP)
    def _():
        for k in range(TPB):
            t = i * TPB + k
            at = attn_ref[pl.ds(t, 1), :]
            g_ref[pl.ds(t, 1), :] = _dot_t(at, w1_ref[:, k * H:(k + 1) * H])

    @pl.when(i >= NTP)
    def _():
        raw = we_ref[0]
        base = (i - NTP) % NSB * SB
        pos = jax.lax.broadcasted_iota(jnp.int32, (SB, T), 0) + base
        starts = st_ref[0]
        ends = en_ref[0]
        mask = ((pos >= starts) & (pos < ends)).astype(jnp.bfloat16)
        hpre = jnp.dot(mask, g_ref[...].astype(jnp.bfloat16),
                       preferred_element_type=jnp.float32) + b1_ref[...]
        h = jnp.maximum(hpre, 0.0).astype(jnp.bfloat16)
        to = (_dot_t(h, w2_ref[...].astype(jnp.bfloat16))
              + b2_ref[...])
        D = H + HF
        mu = (jnp.sum(raw, -1, keepdims=True)
              + jnp.sum(to, -1, keepdims=True)) * (1.0 / D)
        ex2 = (jnp.sum(raw * raw, -1, keepdims=True)
               + jnp.sum(to * to, -1, keepdims=True)) * (1.0 / D)
        r = jax.lax.rsqrt(ex2 - mu * mu + EPS)
        sw = wout_ref[...] * lng_ref[...]
        dr = _dot_t(raw, sw[:, :H])
        dt = _dot_t(to.astype(jnp.bfloat16),
                    sw[:, H:].astype(jnp.bfloat16))
        csum = jnp.sum(sw, axis=1).reshape(1, NL)
        cvec = _dot_t(lnb_ref[...], wout_ref[...]) + bout_ref[...]
        out_ref[0] = r * (dr + dt) - (r * mu) * csum + cvec


@functools.partial(jax.jit, static_argnums=())
def kernel(word_embedding, spans, tag_embedding, in_proj_w, in_proj_b,
           out_proj_w, out_proj_b, W1, b1, W2, b2, ln_g, ln_b, Wout, bout):
    f32 = jnp.float32
    bv = in_proj_b[2 * H:].reshape(1, H)
    opb = out_proj_b.reshape(1, H)
    starts = spans[:, :, 0].astype(jnp.int32).reshape(B, 1, T)
    ends = spans[:, :, 1].astype(jnp.int32).reshape(B, 1, T)

    const = lambda i: (0, 0)
    wv_map = lambda i: (2, 0)
    bmap = lambda i: (jnp.maximum(i - NTP, 0) // NSB, 0, 0)
    rmap = lambda i: (jnp.maximum(i - NTP, 0) // NSB,
                      jnp.maximum(i - NTP, 0) % NSB, 0)

    out = pl.pallas_call(
        _fused_kernel,
        grid=(NTP + B * NSB,),
        in_specs=[
            pl.BlockSpec((T, H), const),
            pl.BlockSpec((H, H), wv_map),
            pl.BlockSpec((1, H), const),
            pl.BlockSpec((H, H), const),
            pl.BlockSpec((1, H), const),
            pl.BlockSpec((H, TPB * H), lambda i: (0, jnp.minimum(i, NTP - 1))),
            pl.BlockSpec((1, SB, H), rmap),
            pl.BlockSpec((1, 1, T), bmap),
            pl.BlockSpec((1, 1, T), bmap),
            pl.BlockSpec((1, H), const),
            pl.BlockSpec((HF, H), const),
            pl.BlockSpec((1, HF), const),
            pl.BlockSpec((1, H + HF), const),
            pl.BlockSpec((1, H + HF), const),
            pl.BlockSpec((NL, H + HF), const),
            pl.BlockSpec((1, NL), const),
        ],
        out_specs=pl.BlockSpec((1, SB, NL), rmap),
        out_shape=jax.ShapeDtypeStruct((B, S, NL), f32),
        scratch_shapes=[pltpu.VMEM((T, H), f32), pltpu.VMEM((T, H), f32)],
    )(tag_embedding.astype(f32), in_proj_w, bv, out_proj_w, opb, W1,
      word_embedding, starts, ends, b1.reshape(1, H), W2,
      b2.reshape(1, HF), ln_g.reshape(1, H + HF), ln_b.reshape(1, H + HF),
      Wout, bout.reshape(1, NL))
    return out


# speedup vs baseline: 16.0381x; 1.0782x over previous
import functools

import jax
import jax.numpy as jnp
from jax.experimental import pallas as pl
from jax.experimental.pallas import tpu as pltpu

B, S, H, T, NH, NL = 4, 512, 1024, 8, 16, 9
HF = 512
EPS = 1e-12
SB = 512
NSB = S // SB
TPB = 2
NTP = T // TPB


def _dot_t(a, b):
    return jax.lax.dot_general(a, b, (((1,), (1,)), ((), ())),
                               preferred_element_type=jnp.float32)


def _fused_kernel(tag_ref, wv_ref, bv_ref, opw_ref, opb_ref, w1_ref,
                  we_ref, st_ref, en_ref, b1_ref, w2_ref, b2_ref,
                  lng_ref, lnb_ref, wout_ref, bout_ref,
                  out_ref, attn_ref, g_ref):
    i = pl.program_id(0)

    @pl.when(i == 0)
    def _():
        vh = _dot_t(tag_ref[...], wv_ref[...]) + bv_ref[...]
        attn_ref[...] = _dot_t(vh, opw_ref[...]) + opb_ref[...]

    @pl.when(i < NTP)
    def _():
        for k in range(TPB):
            t = i * TPB + k
            at = attn_ref[pl.ds(t, 1), :]
            g_ref[pl.ds(t, 1), :] = _dot_t(at, w1_ref[:, k * H:(k + 1) * H])

    @pl.when(i >= NTP)
    def _():
        raw = we_ref[0]
        base = (i - NTP) % NSB * SB
        pos = jax.lax.broadcasted_iota(jnp.int32, (SB, T), 0) + base
        starts = st_ref[0]
        ends = en_ref[0]
        mask = ((pos >= starts) & (pos < ends)).astype(jnp.bfloat16)
        hpre = jnp.dot(mask, g_ref[...].astype(jnp.bfloat16),
                       preferred_element_type=jnp.float32) + b1_ref[...]
        h = jnp.maximum(hpre, 0.0).astype(jnp.bfloat16)
        to = (_dot_t(h, w2_ref[...].astype(jnp.bfloat16))
              + b2_ref[...])
        D = H + HF
        mu = (jnp.sum(raw, -1, keepdims=True)
              + jnp.sum(to, -1, keepdims=True)) * (1.0 / D)
        ex2 = (jnp.sum(raw * raw, -1, keepdims=True)
               + jnp.sum(to * to, -1, keepdims=True)) * (1.0 / D)
        r = jax.lax.rsqrt(ex2 - mu * mu + EPS)
        sw = wout_ref[...] * lng_ref[...]
        dr = _dot_t(raw, sw[:, :H])
        dt = _dot_t(to.astype(jnp.bfloat16),
                    sw[:, H:].astype(jnp.bfloat16))
        csum = jnp.sum(sw, axis=1).reshape(1, NL)
        cvec = _dot_t(lnb_ref[...], wout_ref[...]) + bout_ref[...]
        out_ref[0] = r * (dr + dt) - (r * mu) * csum + cvec


@functools.partial(jax.jit, static_argnums=())
def kernel(word_embedding, spans, tag_embedding, in_proj_w, in_proj_b,
           out_proj_w, out_proj_b, W1, b1, W2, b2, ln_g, ln_b, Wout, bout):
    f32 = jnp.float32
    bv = in_proj_b[2 * H:].reshape(1, H)
    opb = out_proj_b.reshape(1, H)
    starts = spans[:, :, 0].astype(jnp.int32).reshape(B, 1, T)
    ends = spans[:, :, 1].astype(jnp.int32).reshape(B, 1, T)

    const = lambda i: (0, 0)
    wv_map = lambda i: (2, 0)
    bmap = lambda i: (jnp.maximum(i - NTP, 0) // NSB, 0, 0)
    rmap = lambda i: (jnp.maximum(i - NTP, 0) // NSB,
                      jnp.maximum(i - NTP, 0) % NSB, 0)

    out = pl.pallas_call(
        _fused_kernel,
        grid=(NTP + B * NSB,),
        in_specs=[
            pl.BlockSpec((T, H), const),
            pl.BlockSpec((H, H), wv_map),
            pl.BlockSpec((1, H), const),
            pl.BlockSpec((H, H), const),
            pl.BlockSpec((1, H), const),
            pl.BlockSpec((H, TPB * H), lambda i: (0, jnp.minimum(i, NTP - 1))),
            pl.BlockSpec((1, SB, H), rmap),
            pl.BlockSpec((1, 1, T), bmap),
            pl.BlockSpec((1, 1, T), bmap),
            pl.BlockSpec((1, H), const),
            pl.BlockSpec((HF, H), const),
            pl.BlockSpec((1, HF), const),
            pl.BlockSpec((1, H + HF), const),
            pl.BlockSpec((1, H + HF), const),
            pl.BlockSpec((NL, H + HF), const),
            pl.BlockSpec((1, NL), const),
        ],
        out_specs=pl.BlockSpec((1, SB, NL), rmap),
        out_shape=jax.ShapeDtypeStruct((B, S, NL), f32),
        scratch_shapes=[pltpu.VMEM((T, H), f32), pltpu.VMEM((T, H), f32)],
    )(tag_embedding.astype(f32), in_proj_w, bv, out_proj_w, opb, W1,
      word_embedding, starts, ends, b1.reshape(1, H), W2,
      b2.reshape(1, HF), ln_g.reshape(1, H + HF), ln_b.reshape(1, H + HF),
      Wout, bout.reshape(1, NL))
    return out
